# Initial kernel scaffold; baseline (speedup 1.0000x reference)
import jax
import jax.numpy as jnp
from jax.experimental import pallas as pl


def kernel(xyz, num_class, skel_xyz):
    raise NotImplementedError("write your pallas kernel here")


# speedup vs baseline: 10.9379x; 10.9379x over previous
import jax
import jax.numpy as jnp
from jax.experimental import pallas as pl
from jax.experimental.pallas import tpu as pltpu

_R = 512
_N = 4096
_C = 256
_K = 10
_INF = 3.0e38
_BIGI = 1 << 30


def _loss_tile_kernel(xyz_row_ref, xyzT_ref, skel_ref, skelT_ref,
                      out_ref, d_scr, v_scr):
    t = pl.program_id(1)

    rows = xyz_row_ref[0]
    px = rows[:, 0:1]
    py = rows[:, 1:2]
    pz = rows[:, 2:3]
    ax = rows[:, 3:4]
    ay = rows[:, 4:5]
    az = rows[:, 5:6]

    colsT = xyzT_ref[0]
    qx = colsT[0:1, :]
    qy = colsT[1:2, :]
    qz = colsT[2:3, :]
    bx = colsT[3:4, :]
    by = colsT[4:5, :]
    bz = colsT[5:6, :]

    pn = px * px + py * py + pz * pz
    qn = qx * qx + qy * qy + qz * qz
    dot = px * qx + py * qy + pz * qz
    d_scr[...] = pn + qn - 2.0 * dot

    cx = ay * bz - az * by
    cy = az * bx - ax * bz
    cz = ax * by - ay * bx
    c2 = cx * cx + cy * cy + cz * cz
    mx = ax * bx
    my = ay * by
    mz = az * bz
    m2 = mx * mx + my * my + mz * mz
    v_scr[...] = jnp.sqrt(jnp.maximum(jnp.minimum(c2, m2), 1e-12))

    iota = jax.lax.broadcasted_iota(jnp.int32, (_R, _N), 1)
    acc = jnp.zeros((_R, 1), jnp.float32)
    for _ in range(_K):
        d = d_scr[...]
        m = jnp.min(d, axis=1, keepdims=True)
        first = jnp.min(jnp.where(d == m, iota, _BIGI), axis=1,
                        keepdims=True)
        one = iota == first
        acc = acc + jnp.sum(jnp.where(one, v_scr[...], 0.0), axis=1,
                            keepdims=True)
        d_scr[...] = jnp.where(one, _INF, d)

    sT = skelT_ref[0]
    sx = sT[0:1, :]
    sy = sT[1:2, :]
    sz = sT[2:3, :]
    sn = sx * sx + sy * sy + sz * sz
    d2 = pn + sn - 2.0 * (px * sx + py * sy + pz * sz)
    iota2 = jax.lax.broadcasted_iota(jnp.int32, (_R, _C), 1)
    m1 = jnp.min(d2, axis=1, keepdims=True)
    f1 = jnp.min(jnp.where(d2 == m1, iota2, _BIGI), axis=1, keepdims=True)
    m2v = jnp.min(jnp.where(iota2 == f1, _INF, d2), axis=1, keepdims=True)
    voro = jnp.sum(acc * (m2v - m1))

    skl = skel_ref[0]
    kx = skl[:, 0:1]
    ky = skl[:, 1:2]
    kz = skl[:, 2:3]
    kn = kx * kx + ky * ky + kz * kz
    d3 = kn + sn - 2.0 * (kx * sx + ky * sy + kz * sz)
    iota3 = jax.lax.broadcasted_iota(jnp.int32, (_C, _C), 1)
    m1c = jnp.min(d3, axis=1, keepdims=True)
    f1c = jnp.min(jnp.where(d3 == m1c, iota3, _BIGI), axis=1, keepdims=True)
    m2c = jnp.min(jnp.where(iota3 == f1c, _INF, d3), axis=1, keepdims=True)
    chosen = jnp.sum(jnp.sqrt(jnp.maximum(m2c, 1e-12)))
    chosen = jnp.where(t == 0, chosen, 0.0)

    out_ref[0, 0, 0] = voro - 0.5 * chosen


def kernel(xyz, num_class, skel_xyz):
    B = xyz.shape[0]
    T = _N // _R
    xyzT = jnp.transpose(xyz, (0, 2, 1))
    skelT = jnp.transpose(skel_xyz, (0, 2, 1))

    parts = pl.pallas_call(
        _loss_tile_kernel,
        grid=(B, T),
        in_specs=[
            pl.BlockSpec((1, _R, 6), lambda b, t: (b, t, 0)),
            pl.BlockSpec((1, 6, _N), lambda b, t: (b, 0, 0)),
            pl.BlockSpec((1, _C, 3), lambda b, t: (b, 0, 0)),
            pl.BlockSpec((1, 3, _C), lambda b, t: (b, 0, 0)),
        ],
        out_specs=pl.BlockSpec((1, 1, 1), lambda b, t: (b * T + t, 0, 0),
                               memory_space=pltpu.SMEM),
        out_shape=jax.ShapeDtypeStruct((B * T, 1, 1), jnp.float32),
        scratch_shapes=[
            pltpu.VMEM((_R, _N), jnp.float32),
            pltpu.VMEM((_R, _N), jnp.float32),
        ],
    )(xyz, xyzT, skel_xyz, skelT)
    return jnp.sum(parts)


# speedup vs baseline: 19.1366x; 1.7496x over previous
import jax
import jax.numpy as jnp
from jax.experimental import pallas as pl
from jax.experimental.pallas import tpu as pltpu

_R = 512
_N = 4096
_C = 256
_K = 10
_INF = 3.0e38
_BIGI = 1 << 30


def _loss_tile_kernel(xyz_row_ref, xyzT_ref, skel_ref, skelT_ref,
                      out_ref, d_scr, v_scr):
    t = pl.program_id(1)

    rows = xyz_row_ref[0]
    px = rows[:, 0:1]
    py = rows[:, 1:2]
    pz = rows[:, 2:3]
    ax = rows[:, 3:4]
    ay = rows[:, 4:5]
    az = rows[:, 5:6]

    colsT = xyzT_ref[0]
    qx = colsT[0:1, :]
    qy = colsT[1:2, :]
    qz = colsT[2:3, :]
    bx = colsT[3:4, :]
    by = colsT[4:5, :]
    bz = colsT[5:6, :]

    iota = jax.lax.broadcasted_iota(jnp.int32, (_R, _N), 1)
    row_id = t * _R + jax.lax.broadcasted_iota(jnp.int32, (_R, 1), 0)

    pn = px * px + py * py + pz * pz
    qn = qx * qx + qy * qy + qz * qz
    dot = px * qx + py * qy + pz * qz
    d_scr[...] = jnp.where(iota == row_id, _INF, pn + qn - 2.0 * dot)

    cx = ay * bz - az * by
    cy = az * bx - ax * bz
    cz = ax * by - ay * bx
    c2 = cx * cx + cy * cy + cz * cz
    mx = ax * bx
    my = ay * by
    mz = az * bz
    m2 = mx * mx + my * my + mz * mz
    v_scr[...] = jnp.sqrt(jnp.maximum(jnp.minimum(c2, m2), 1e-12))

    acc = jnp.full((_R, 1), 1e-6, jnp.float32)
    for it in range(_K - 1):
        d = d_scr[...]
        m = jnp.min(d, axis=1, keepdims=True)
        eq = d == m
        acc = acc + jnp.sum(jnp.where(eq, v_scr[...], 0.0), axis=1,
                            keepdims=True)
        if it != _K - 2:
            d_scr[...] = jnp.where(eq, _INF, d)

    sT = skelT_ref[0]
    sx = sT[0:1, :]
    sy = sT[1:2, :]
    sz = sT[2:3, :]
    sn = sx * sx + sy * sy + sz * sz
    d2 = pn + sn - 2.0 * (px * sx + py * sy + pz * sz)
    iota2 = jax.lax.broadcasted_iota(jnp.int32, (_R, _C), 1)
    m1 = jnp.min(d2, axis=1, keepdims=True)
    f1 = jnp.min(jnp.where(d2 == m1, iota2, _BIGI), axis=1, keepdims=True)
    m2v = jnp.min(jnp.where(iota2 == f1, _INF, d2), axis=1, keepdims=True)
    voro = jnp.sum(acc * (m2v - m1))

    skl = skel_ref[0]
    kx = skl[:, 0:1]
    ky = skl[:, 1:2]
    kz = skl[:, 2:3]
    kn = kx * kx + ky * ky + kz * kz
    d3 = kn + sn - 2.0 * (kx * sx + ky * sy + kz * sz)
    iota3 = jax.lax.broadcasted_iota(jnp.int32, (_C, _C), 1)
    m1c = jnp.min(d3, axis=1, keepdims=True)
    f1c = jnp.min(jnp.where(d3 == m1c, iota3, _BIGI), axis=1, keepdims=True)
    m2c = jnp.min(jnp.where(iota3 == f1c, _INF, d3), axis=1, keepdims=True)
    chosen = jnp.sum(jnp.sqrt(jnp.maximum(m2c, 1e-12)))
    chosen = jnp.where(t == 0, chosen, 0.0)

    out_ref[0, 0, 0] = voro - 0.5 * chosen


def kernel(xyz, num_class, skel_xyz):
    B = xyz.shape[0]
    T = _N // _R
    xyzT = jnp.transpose(xyz, (0, 2, 1))
    skelT = jnp.transpose(skel_xyz, (0, 2, 1))

    parts = pl.pallas_call(
        _loss_tile_kernel,
        grid=(B, T),
        in_specs=[
            pl.BlockSpec((1, _R, 6), lambda b, t: (b, t, 0)),
            pl.BlockSpec((1, 6, _N), lambda b, t: (b, 0, 0)),
            pl.BlockSpec((1, _C, 3), lambda b, t: (b, 0, 0)),
            pl.BlockSpec((1, 3, _C), lambda b, t: (b, 0, 0)),
        ],
        out_specs=pl.BlockSpec((1, 1, 1), lambda b, t: (b * T + t, 0, 0),
                               memory_space=pltpu.SMEM),
        out_shape=jax.ShapeDtypeStruct((B * T, 1, 1), jnp.float32),
        scratch_shapes=[
            pltpu.VMEM((_R, _N), jnp.float32),
            pltpu.VMEM((_R, _N), jnp.float32),
        ],
        compiler_params=pltpu.CompilerParams(
            dimension_semantics=("parallel", "parallel"),
        ),
    )(xyz, xyzT, skel_xyz, skelT)
    return jnp.sum(parts)


# speedup vs baseline: 27.9483x; 1.4605x over previous
import jax
import jax.numpy as jnp
from jax.experimental import pallas as pl
from jax.experimental.pallas import tpu as pltpu

_R = 512
_N = 4096
_C = 256
_K = 10
_S = 8
_W = _N // _S
_INF = 3.0e38
_BIGI = 1 << 30


def _loss_tile_kernel(xyz_row_ref, xyzT_ref, skel_ref, skelT_ref,
                      out_ref, dc_scr, vc_scr):
    t = pl.program_id(1)

    rows = xyz_row_ref[0]
    px = rows[:, 0:1]
    py = rows[:, 1:2]
    pz = rows[:, 2:3]
    ax = rows[:, 3:4]
    ay = rows[:, 4:5]
    az = rows[:, 5:6]
    colsT = xyzT_ref[0]

    pn = px * px + py * py + pz * pz
    ax2 = ax * ax
    ay2 = ay * ay
    az2 = az * az
    an = ax2 + ay2 + az2

    row_id = t * _R + jax.lax.broadcasted_iota(jnp.int32, (_R, 1), 0)
    iota_w = jax.lax.broadcasted_iota(jnp.int32, (_R, _W), 1)

    m1 = jnp.full((_R, _W), _INF, jnp.float32)
    m2 = jnp.full((_R, _W), _INF, jnp.float32)
    m3 = jnp.full((_R, _W), _INF, jnp.float32)
    vm1 = jnp.zeros((_R, _W), jnp.float32)
    vm2 = jnp.zeros((_R, _W), jnp.float32)
    vm3 = jnp.zeros((_R, _W), jnp.float32)
    for s in range(_S):
        qx = colsT[0:1, s * _W:(s + 1) * _W]
        qy = colsT[1:2, s * _W:(s + 1) * _W]
        qz = colsT[2:3, s * _W:(s + 1) * _W]
        bx = colsT[3:4, s * _W:(s + 1) * _W]
        by = colsT[4:5, s * _W:(s + 1) * _W]
        bz = colsT[5:6, s * _W:(s + 1) * _W]
        qn = qx * qx + qy * qy + qz * qz
        bx2 = bx * bx
        by2 = by * by
        bz2 = bz * bz
        bn = bx2 + by2 + bz2

        dot = px * qx + py * qy + pz * qz
        d = pn + qn - 2.0 * dot
        d = jnp.where(iota_w + s * _W == row_id, _INF, d)

        g = ax * bx + ay * by + az * bz
        c2 = an * bn - g * g
        mm = ax2 * bx2 + ay2 * by2 + az2 * bz2
        v = jnp.sqrt(jnp.maximum(jnp.minimum(c2, mm), 1e-12))

        sel1 = d < m1
        b1 = jnp.maximum(m1, d)
        vb1 = jnp.where(sel1, vm1, v)
        m1 = jnp.minimum(m1, d)
        vm1 = jnp.where(sel1, v, vm1)
        sel2 = b1 < m2
        b2 = jnp.maximum(m2, b1)
        vb2 = jnp.where(sel2, vm2, vb1)
        m2 = jnp.minimum(m2, b1)
        vm2 = jnp.where(sel2, vb1, vm2)
        sel3 = b2 < m3
        m3 = jnp.where(sel3, b2, m3)
        vm3 = jnp.where(sel3, vb2, vm3)

    dc_scr[:, 0:_W] = m1
    dc_scr[:, _W:2 * _W] = m2
    dc_scr[:, 2 * _W:3 * _W] = m3
    vc_scr[:, 0:_W] = vm1
    vc_scr[:, _W:2 * _W] = vm2
    vc_scr[:, 2 * _W:3 * _W] = vm3

    acc = jnp.full((_R, 1), 1e-6, jnp.float32)
    for it in range(_K - 1):
        d = dc_scr[...]
        m = jnp.min(d, axis=1, keepdims=True)
        eq = d == m
        acc = acc + jnp.sum(jnp.where(eq, vc_scr[...], 0.0), axis=1,
                            keepdims=True)
        if it != _K - 2:
            dc_scr[...] = jnp.where(eq, _INF, d)

    sT = skelT_ref[0]
    sx = sT[0:1, :]
    sy = sT[1:2, :]
    sz = sT[2:3, :]
    sn = sx * sx + sy * sy + sz * sz
    d2 = pn + sn - 2.0 * (px * sx + py * sy + pz * sz)
    iota2 = jax.lax.broadcasted_iota(jnp.int32, (_R, _C), 1)
    m1v = jnp.min(d2, axis=1, keepdims=True)
    f1 = jnp.min(jnp.where(d2 == m1v, iota2, _BIGI), axis=1, keepdims=True)
    m2v = jnp.min(jnp.where(iota2 == f1, _INF, d2), axis=1, keepdims=True)
    voro = jnp.sum(acc * (m2v - m1v))

    skl = skel_ref[0]
    kx = skl[:, 0:1]
    ky = skl[:, 1:2]
    kz = skl[:, 2:3]
    kn = kx * kx + ky * ky + kz * kz
    d3 = kn + sn - 2.0 * (kx * sx + ky * sy + kz * sz)
    iota3 = jax.lax.broadcasted_iota(jnp.int32, (_C, _C), 1)
    m1c = jnp.min(d3, axis=1, keepdims=True)
    f1c = jnp.min(jnp.where(d3 == m1c, iota3, _BIGI), axis=1, keepdims=True)
    m2c = jnp.min(jnp.where(iota3 == f1c, _INF, d3), axis=1, keepdims=True)
    chosen = jnp.sum(jnp.sqrt(jnp.maximum(m2c, 1e-12)))
    chosen = jnp.where(t == 0, chosen, 0.0)

    out_ref[0, 0, 0] = voro - 0.5 * chosen


def kernel(xyz, num_class, skel_xyz):
    B = xyz.shape[0]
    T = _N // _R
    xyzT = jnp.transpose(xyz, (0, 2, 1))
    skelT = jnp.transpose(skel_xyz, (0, 2, 1))

    parts = pl.pallas_call(
        _loss_tile_kernel,
        grid=(B, T),
        in_specs=[
            pl.BlockSpec((1, _R, 6), lambda b, t: (b, t, 0)),
            pl.BlockSpec((1, 6, _N), lambda b, t: (b, 0, 0)),
            pl.BlockSpec((1, _C, 3), lambda b, t: (b, 0, 0)),
            pl.BlockSpec((1, 3, _C), lambda b, t: (b, 0, 0)),
        ],
        out_specs=pl.BlockSpec((1, 1, 1), lambda b, t: (b * T + t, 0, 0),
                               memory_space=pltpu.SMEM),
        out_shape=jax.ShapeDtypeStruct((B * T, 1, 1), jnp.float32),
        scratch_shapes=[
            pltpu.VMEM((_R, 3 * _W), jnp.float32),
            pltpu.VMEM((_R, 3 * _W), jnp.float32),
        ],
        compiler_params=pltpu.CompilerParams(
            dimension_semantics=("parallel", "parallel"),
        ),
    )(xyz, xyzT, skel_xyz, skelT)
    return jnp.sum(parts)


# speedup vs baseline: 34.1769x; 1.2229x over previous
import jax
import jax.numpy as jnp
from jax.experimental import pallas as pl
from jax.experimental.pallas import tpu as pltpu

_R = 512
_N = 4096
_C = 256
_K = 10
_S = 8
_W = _N // _S
_INF = 3.0e38
_BIGI = 1 << 30


def _loss_tile_kernel(xyz_row_ref, xyzT_ref, skel_ref, skelT_ref, out_ref):
    t = pl.program_id(1)

    rows = xyz_row_ref[0]
    px = rows[:, 0:1]
    py = rows[:, 1:2]
    pz = rows[:, 2:3]
    ax = rows[:, 3:4]
    ay = rows[:, 4:5]
    az = rows[:, 5:6]
    colsT = xyzT_ref[0]

    pn = px * px + py * py + pz * pz
    ax2 = ax * ax
    ay2 = ay * ay
    az2 = az * az
    an = ax2 + ay2 + az2

    m1 = jnp.full((_R, _W), _INF, jnp.float32)
    m2 = jnp.full((_R, _W), _INF, jnp.float32)
    m3 = jnp.full((_R, _W), _INF, jnp.float32)
    vm1 = jnp.zeros((_R, _W), jnp.float32)
    vm2 = jnp.zeros((_R, _W), jnp.float32)
    vm3 = jnp.zeros((_R, _W), jnp.float32)
    for s in range(_S):
        qx = colsT[0:1, s * _W:(s + 1) * _W]
        qy = colsT[1:2, s * _W:(s + 1) * _W]
        qz = colsT[2:3, s * _W:(s + 1) * _W]
        bx = colsT[3:4, s * _W:(s + 1) * _W]
        by = colsT[4:5, s * _W:(s + 1) * _W]
        bz = colsT[5:6, s * _W:(s + 1) * _W]
        qn = qx * qx + qy * qy + qz * qz
        bx2 = bx * bx
        by2 = by * by
        bz2 = bz * bz
        bn = bx2 + by2 + bz2

        dot = px * qx + py * qy + pz * qz
        d = pn + qn - 2.0 * dot

        g = ax * bx + ay * by + az * bz
        c2 = an * bn - g * g
        mm = ax2 * bx2 + ay2 * by2 + az2 * bz2
        v = jnp.minimum(c2, mm)

        sel1 = d < m1
        b1 = jnp.maximum(m1, d)
        vb1 = jnp.where(sel1, vm1, v)
        m1 = jnp.minimum(m1, d)
        vm1 = jnp.where(sel1, v, vm1)
        sel2 = b1 < m2
        b2 = jnp.maximum(m2, b1)
        vb2 = jnp.where(sel2, vm2, vb1)
        m2 = jnp.minimum(m2, b1)
        vm2 = jnp.where(sel2, vb1, vm2)
        sel3 = b2 < m3
        m3 = jnp.where(sel3, b2, m3)
        vm3 = jnp.where(sel3, vb2, vm3)

    dcand = jnp.concatenate([m1, m2, m3], axis=1)
    for _ in range(_K):
        m = jnp.min(dcand, axis=1, keepdims=True)
        dcand = jnp.where(dcand == m, _INF, dcand)
    vcand = jnp.concatenate([vm1, vm2, vm3], axis=1)
    vcand = jnp.sqrt(jnp.maximum(vcand, 1e-12))
    acc = jnp.sum(jnp.where(dcand == _INF, vcand, 0.0), axis=1,
                  keepdims=True)

    sT = skelT_ref[0]
    sx = sT[0:1, :]
    sy = sT[1:2, :]
    sz = sT[2:3, :]
    sn = sx * sx + sy * sy + sz * sz
    d2 = pn + sn - 2.0 * (px * sx + py * sy + pz * sz)
    iota2 = jax.lax.broadcasted_iota(jnp.int32, (_R, _C), 1)
    m1v = jnp.min(d2, axis=1, keepdims=True)
    f1 = jnp.min(jnp.where(d2 == m1v, iota2, _BIGI), axis=1, keepdims=True)
    m2v = jnp.min(jnp.where(iota2 == f1, _INF, d2), axis=1, keepdims=True)
    voro = jnp.sum(acc * (m2v - m1v))

    skl = skel_ref[0]
    kx = skl[:, 0:1]
    ky = skl[:, 1:2]
    kz = skl[:, 2:3]
    kn = kx * kx + ky * ky + kz * kz
    d3 = kn + sn - 2.0 * (kx * sx + ky * sy + kz * sz)
    iota3 = jax.lax.broadcasted_iota(jnp.int32, (_C, _C), 1)
    m1c = jnp.min(d3, axis=1, keepdims=True)
    f1c = jnp.min(jnp.where(d3 == m1c, iota3, _BIGI), axis=1, keepdims=True)
    m2c = jnp.min(jnp.where(iota3 == f1c, _INF, d3), axis=1, keepdims=True)
    chosen = jnp.sum(jnp.sqrt(jnp.maximum(m2c, 1e-12)))
    chosen = jnp.where(t == 0, chosen, 0.0)

    out_ref[0, 0, 0] = voro - 0.5 * chosen


def kernel(xyz, num_class, skel_xyz):
    B = xyz.shape[0]
    T = _N // _R
    xyzT = jnp.transpose(xyz, (0, 2, 1))
    skelT = jnp.transpose(skel_xyz, (0, 2, 1))

    parts = pl.pallas_call(
        _loss_tile_kernel,
        grid=(B, T),
        in_specs=[
            pl.BlockSpec((1, _R, 6), lambda b, t: (b, t, 0)),
            pl.BlockSpec((1, 6, _N), lambda b, t: (b, 0, 0)),
            pl.BlockSpec((1, _C, 3), lambda b, t: (b, 0, 0)),
            pl.BlockSpec((1, 3, _C), lambda b, t: (b, 0, 0)),
        ],
        out_specs=pl.BlockSpec((1, 1, 1), lambda b, t: (b * T + t, 0, 0),
                               memory_space=pltpu.SMEM),
        out_shape=jax.ShapeDtypeStruct((B * T, 1, 1), jnp.float32),
        compiler_params=pltpu.CompilerParams(
            dimension_semantics=("parallel", "parallel"),
        ),
    )(xyz, xyzT, skel_xyz, skelT)
    return jnp.sum(parts)
